# baseline (speedup 1.0000x reference)
import numpy as np
import jax
import jax.numpy as jnp
from jax.experimental import pallas as pl
from jax.experimental.pallas import tpu as pltpu

_LEVELS = np.array([8, 8, 8, 5, 5, 5], dtype=np.int64)
_D = 6
_EPS = 1e-3

_HALF_L = (_LEVELS.astype(np.float64) - 1.0) * (1.0 - _EPS) / 2.0
_OFFSET = np.where(_LEVELS % 2 == 0, 0.5, 0.0)
_SHIFT = np.arctanh(_OFFSET / _HALF_L)
_HALF_W = (_LEVELS // 2).astype(np.float64)
_BASIS = np.concatenate([[1], np.cumprod(_LEVELS[:-1])]).astype(np.float64)
_IDX_OFFSET = float(np.sum(_HALF_W * _BASIS))

_N_TOK = 9216
_G = 768
_IN_SIZES = [768, 1536, 2304, 4608]
_OUT_SIZES = [2304, 2304, 2304, 1536, 768]
assert sum(_IN_SIZES) == _N_TOK
assert sum(_OUT_SIZES) == _N_TOK
assert all(v % _G == 0 for v in _IN_SIZES + _OUT_SIZES)
_NC = _N_TOK // _G


def _fsq_body(x_hbm, w_in_t_ref, w_out_ref, b_out_ref, consts_ref,
              q_x_hbm, idx_ref, x_buf, q_buf, in_sems, out_sems):
    in_starts = [int(v) for v in np.cumsum([0] + _IN_SIZES)[:-1]]

    def in_dma(k):
        st, sz = in_starts[k], _IN_SIZES[k]
        return pltpu.make_async_copy(
            x_hbm.at[pl.ds(st, sz)], x_buf.at[pl.ds(st, sz)], in_sems.at[k])

    out_starts = [int(v) for v in np.cumsum([0] + _OUT_SIZES)[:-1]]

    def out_dma(k):
        st, sz = out_starts[k], _OUT_SIZES[k]
        return pltpu.make_async_copy(
            q_buf.at[pl.ds(st, sz)], q_x_hbm.at[pl.ds(st, sz)],
            out_sems.at[k])

    for k in range(len(_IN_SIZES)):
        in_dma(k).start()

    half_l = consts_ref[:, 0:1]
    offset = consts_ref[:, 1:2]
    shift = consts_ref[:, 2:3]
    inv_half_w = consts_ref[:, 3:4]
    basis = consts_ref[:, 4:5]
    b_in = consts_ref[:, 5:6]

    covered = 0
    next_in = 0
    next_out = 0
    for c in range(_NC):
        st = c * _G
        while covered < st + _G:
            in_dma(next_in).wait()
            covered += _IN_SIZES[next_in]
            next_in += 1
        z_t = jax.lax.dot_general(
            w_in_t_ref[...], x_buf[st:st + _G], (((1,), (1,)), ((), ())),
            preferred_element_type=jnp.float32) + b_in
        bounded = jnp.tanh(z_t + shift) * half_l - offset
        q = jnp.round(bounded)
        codes_t = q * inv_half_w
        idx = jnp.sum(q * basis, axis=0) + _IDX_OFFSET
        idx_ref[c] = idx.astype(jnp.int32).reshape(1, _G)
        q_x = jax.lax.dot_general(
            codes_t, w_out_ref[...], (((0,), (0,)), ((), ())),
            preferred_element_type=jnp.float32)
        q_buf[st:st + _G] = q_x + b_out_ref[...]

    for k in range(len(_OUT_SIZES)):
        out_dma(k).start()
    for k in range(len(_OUT_SIZES)):
        out_dma(k).wait()


@jax.jit
def _fsq(x, W_in, b_in, W_out, b_out):
    B, T, C = x.shape
    n_tok = B * T
    x2 = x.reshape(n_tok, C)
    consts = jnp.asarray(
        np.stack([_HALF_L, _OFFSET, _SHIFT, 1.0 / _HALF_W, _BASIS,
                  np.zeros(_D)], axis=1),
        dtype=jnp.float32)
    consts = consts.at[:, 5].set(b_in)
    w_in_t = W_in.T

    q_x, idx = pl.pallas_call(
        _fsq_body,
        in_specs=[
            pl.BlockSpec(memory_space=pl.ANY),
            pl.BlockSpec(memory_space=pltpu.VMEM),
            pl.BlockSpec(memory_space=pltpu.VMEM),
            pl.BlockSpec(memory_space=pltpu.VMEM),
            pl.BlockSpec(memory_space=pltpu.VMEM),
        ],
        out_specs=[
            pl.BlockSpec(memory_space=pl.ANY),
            pl.BlockSpec(memory_space=pltpu.VMEM),
        ],
        out_shape=[
            jax.ShapeDtypeStruct((n_tok, C), jnp.float32),
            jax.ShapeDtypeStruct((_NC, 1, _G), jnp.int32),
        ],
        scratch_shapes=[
            pltpu.VMEM((n_tok, C), jnp.float32),
            pltpu.VMEM((n_tok, C), jnp.float32),
            pltpu.SemaphoreType.DMA((len(_IN_SIZES),)),
            pltpu.SemaphoreType.DMA((len(_OUT_SIZES),)),
        ],
    )(x2, w_in_t, W_out, b_out.reshape(1, C), consts)

    return q_x.reshape(B, T, C), idx.reshape(B, T)


def kernel(x, W_in, b_in, W_out, b_out):
    return _fsq(x, W_in, b_in, W_out, b_out)


# speedup vs baseline: 1.2063x; 1.2063x over previous
import functools

import numpy as np
import jax
import jax.numpy as jnp
from jax.experimental import pallas as pl
from jax.experimental.pallas import tpu as pltpu

_LEVELS = np.array([8, 8, 8, 5, 5, 5], dtype=np.int64)
_D = 6
_EPS = 1e-3

_HALF_L = (_LEVELS.astype(np.float64) - 1.0) * (1.0 - _EPS) / 2.0
_OFFSET = np.where(_LEVELS % 2 == 0, 0.5, 0.0)
_SHIFT = np.arctanh(_OFFSET / _HALF_L)
_HALF_W = (_LEVELS // 2).astype(np.float64)
_BASIS = np.concatenate([[1], np.cumprod(_LEVELS[:-1])]).astype(np.float64)
_IDX_OFFSET = float(np.sum(_HALF_W * _BASIS))


def _fsq_body(x_ref, w_in_t_ref, w_out_ref, consts_ref,
              q_x_ref, idx_ref):
    half_l = consts_ref[:, 0:1]
    offset = consts_ref[:, 1:2]
    shift = consts_ref[:, 2:3]
    inv_half_w = consts_ref[:, 3:4]
    basis = consts_ref[:, 4:5]
    b_in = consts_ref[:, 5:6]

    z_t = jax.lax.dot_general(
        w_in_t_ref[...], x_ref[...], (((1,), (1,)), ((), ())),
        preferred_element_type=jnp.float32) + b_in
    bounded = jnp.tanh(z_t + shift) * half_l - offset
    q = jnp.round(bounded)
    codes_t = q * inv_half_w
    idx = jnp.sum(q * basis, axis=0) + _IDX_OFFSET
    idx_ref[pl.program_id(0)] = idx.astype(jnp.int32).reshape(1, idx.shape[0])
    codes_aug = jnp.concatenate(
        [codes_t, jnp.ones((1, codes_t.shape[1]), jnp.float32)], axis=0)
    q_x_ref[...] = jax.lax.dot_general(
        codes_aug, w_out_ref[...], (((0,), (0,)), ((), ())),
        preferred_element_type=jnp.float32)


@jax.jit
def _fsq(x, W_in, b_in, W_out, b_out):
    B, T, C = x.shape
    n_tok = B * T
    x2 = x.reshape(n_tok, C)
    blk = 3072
    grid = (n_tok // blk,)
    consts = jnp.asarray(
        np.stack([_HALF_L, _OFFSET, _SHIFT, 1.0 / _HALF_W, _BASIS,
                  np.zeros(_D)], axis=1),
        dtype=jnp.float32)
    consts = consts.at[:, 5].set(b_in)
    w_in_t = W_in.T
    w_out_aug = jnp.concatenate([W_out, b_out.reshape(1, C)], axis=0)

    q_x, idx = pl.pallas_call(
        _fsq_body,
        grid=grid,
        in_specs=[
            pl.BlockSpec((blk, C), lambda i: (i, 0)),
            pl.BlockSpec((_D, C), lambda i: (0, 0)),
            pl.BlockSpec((_D + 1, C), lambda i: (0, 0)),
            pl.BlockSpec((_D, 6), lambda i: (0, 0)),
        ],
        out_specs=[
            pl.BlockSpec((blk, C), lambda i: (i, 0)),
            pl.BlockSpec((n_tok // blk, 1, blk), lambda i: (0, 0, 0)),
        ],
        out_shape=[
            jax.ShapeDtypeStruct((n_tok, C), jnp.float32),
            jax.ShapeDtypeStruct((n_tok // blk, 1, blk), jnp.int32),
        ],
        compiler_params=pltpu.CompilerParams(
            dimension_semantics=("parallel",)),
    )(x2, w_in_t, w_out_aug, consts)

    return q_x.reshape(B, T, C), idx.reshape(B, T)


def kernel(x, W_in, b_in, W_out, b_out):
    return _fsq(x, W_in, b_in, W_out, b_out)


# speedup vs baseline: 1.2764x; 1.0581x over previous
import functools

import numpy as np
import jax
import jax.numpy as jnp
from jax.experimental import pallas as pl
from jax.experimental.pallas import tpu as pltpu

_LEVELS = np.array([8, 8, 8, 5, 5, 5], dtype=np.int64)
_D = 6
_EPS = 1e-3

_HALF_L = (_LEVELS.astype(np.float64) - 1.0) * (1.0 - _EPS) / 2.0
_OFFSET = np.where(_LEVELS % 2 == 0, 0.5, 0.0)
_SHIFT = np.arctanh(_OFFSET / _HALF_L)
_HALF_W = (_LEVELS // 2).astype(np.float64)
_BASIS = np.concatenate([[1], np.cumprod(_LEVELS[:-1])]).astype(np.float64)
_IDX_OFFSET = float(np.sum(_HALF_W * _BASIS))


def _fsq_body(x_ref, w_in_t_ref, w_out_ref, b_out_ref, consts_ref,
              q_x_ref, idx_ref):
    half_l = consts_ref[:, 0:1]
    offset = consts_ref[:, 1:2]
    shift = consts_ref[:, 2:3]
    inv_half_w = consts_ref[:, 3:4]
    basis = consts_ref[:, 4:5]
    b_in = consts_ref[:, 5:6]

    z_t = jax.lax.dot_general(
        w_in_t_ref[...], x_ref[...], (((1,), (1,)), ((), ())),
        preferred_element_type=jnp.float32) + b_in
    bounded = jnp.tanh(z_t + shift) * half_l - offset
    q = jnp.round(bounded)
    codes_t = q * inv_half_w
    idx = jnp.sum(q * basis, axis=0) + _IDX_OFFSET
    idx_ref[pl.program_id(0)] = idx.astype(jnp.int32).reshape(1, idx.shape[0])
    q_x = jax.lax.dot_general(
        codes_t, w_out_ref[...], (((0,), (0,)), ((), ())),
        preferred_element_type=jnp.float32)
    q_x_ref[...] = q_x + b_out_ref[...]


@jax.jit
def _fsq(x, W_in, b_in, W_out, b_out):
    B, T, C = x.shape
    n_tok = B * T
    x2 = x.reshape(n_tok, C)
    blk = 3072
    grid = (n_tok // blk,)
    consts = jnp.asarray(
        np.stack([_HALF_L, _OFFSET, _SHIFT, 1.0 / _HALF_W, _BASIS,
                  np.zeros(_D)], axis=1),
        dtype=jnp.float32)
    consts = consts.at[:, 5].set(b_in)
    w_in_t = W_in.T

    q_x, idx = pl.pallas_call(
        _fsq_body,
        grid=grid,
        in_specs=[
            pl.BlockSpec((blk, C), lambda i: (i, 0)),
            pl.BlockSpec((_D, C), lambda i: (0, 0)),
            pl.BlockSpec((_D, C), lambda i: (0, 0)),
            pl.BlockSpec((1, C), lambda i: (0, 0)),
            pl.BlockSpec((_D, 6), lambda i: (0, 0)),
        ],
        out_specs=[
            pl.BlockSpec((blk, C), lambda i: (i, 0)),
            pl.BlockSpec((n_tok // blk, 1, blk), lambda i: (0, 0, 0)),
        ],
        out_shape=[
            jax.ShapeDtypeStruct((n_tok, C), jnp.float32),
            jax.ShapeDtypeStruct((n_tok // blk, 1, blk), jnp.int32),
        ],
        compiler_params=pltpu.CompilerParams(
            dimension_semantics=("parallel",)),
    )(x2, w_in_t, W_out, b_out.reshape(1, C), consts)

    return q_x.reshape(B, T, C), idx.reshape(B, T)


def kernel(x, W_in, b_in, W_out, b_out):
    return _fsq(x, W_in, b_in, W_out, b_out)
